# baseline (speedup 1.0000x reference)
import jax
import jax.numpy as jnp
from jax.experimental import pallas as pl
from jax.experimental.pallas import tpu as pltpu

_T = 13
_COU_EMBED = 32
_DN = (((0,), (0,)), ((), ()))


_BB = 2


def _main_body(sidx_ref,
               sd_ref, ed_ref, pt_ref, dt_ref, m_ref, a_ref,
               sdn_ref, edn_ref, vT_ref, vpt_ref, vdt_ref, vft_ref,
               vnum_ref, dm_ref, wn_ref, we_ref,
               node_out_ref, edge_hbm, escratch, esem):
    g = pl.program_id(0)
    n = node_out_ref.shape[2]
    f32 = jnp.float32

    we = we_ref[...]
    wn = wn_ref[...]
    iota_s = jax.lax.broadcasted_iota(jnp.int32, (n, 1), 0)
    iota_l = jax.lax.broadcasted_iota(jnp.int32, (1, n), 1)

    for bb in range(_BB):
        sd = sd_ref[bb]
        ed = ed_ref[bb]
        pt = pt_ref[bb]
        dt = dt_ref[bb]
        sdn = sdn_ref[bb]
        edn = edn_ref[bb]
        vT = jnp.transpose(vT_ref[bb])
        vpt = vpt_ref[bb]
        vdt = vdt_ref[bb]
        vft = vft_ref[bb]

        for t in range(_T):
            slot = bb * _T + t

            @pl.when(g > 0)
            def _reclaim(slot=slot, bb=bb, t=t):
                pltpu.make_async_copy(
                    escratch.at[slot],
                    edge_hbm.at[(g - 1) * _BB + bb, t],
                    esem.at[slot]).wait()

            m = m_ref[bb, t]
            x = jnp.concatenate(
                [sd * m, ed * m, pt * m, dt * m, a_ref[bb, t]], axis=0)
            escratch[slot] = jax.lax.dot_general(
                x, we, _DN, preferred_element_type=f32)
            pltpu.make_async_copy(
                escratch.at[slot],
                edge_hbm.at[g * _BB + bb, t],
                esem.at[slot]).start()

            s = sidx_ref[g * _BB + bb, t]
            ohs = (iota_s == s).astype(f32)
            ohl = (iota_l == s).astype(f32)
            ed_row = jnp.sum(edn * ohs, axis=0, keepdims=True)
            sd_row = jnp.sum(sdn * ohs, axis=0, keepdims=True)
            t_c = jnp.sum(vft * ohl)
            dm = dm_ref[bb, t]
            xn = jnp.concatenate([
                vT,
                vpt - t_c,
                t_c - vdt,
                ed_row * dm,
                sd_row * dm,
                vnum_ref[bb, t],
            ], axis=0) * dm
            node_out_ref[bb, t] = jax.lax.dot_general(
                xn, wn, _DN, preferred_element_type=f32)

    @pl.when(g == pl.num_programs(0) - 1)
    def _drain():
        for bb in range(_BB):
            for t in range(_T):
                slot = bb * _T + t
                pltpu.make_async_copy(
                    escratch.at[slot],
                    edge_hbm.at[g * _BB + bb, t],
                    esem.at[slot]).wait()


def _embed_body(ids_ref, table_hbm, cou_ref, out_ref, row_vmem, sem):
    b = pl.program_id(0)
    idx = ids_ref[b]
    base = pl.multiple_of((idx // 128) * 128, 128)
    cp = pltpu.make_async_copy(
        table_hbm.at[:, pl.ds(base, 128)], row_vmem, sem)
    cp.start()
    cp.wait()
    off = idx % 128
    ohl = (jax.lax.broadcasted_iota(jnp.int32, (1, 128), 1) == off
           ).astype(jnp.float32)
    row = jnp.sum(row_vmem[...] * ohl, axis=1, keepdims=True)
    t = out_ref.shape[1]
    out_ref[0, :, 0:_COU_EMBED] = jnp.broadcast_to(
        jnp.transpose(row), (t, _COU_EMBED))
    out_ref[0, :, _COU_EMBED:_COU_EMBED + 3] = jnp.broadcast_to(
        cou_ref[0][:, 1:4], (t, 3))


def kernel(E_ed, V, V_reach_mask, V_pt, E_sd, V_ft, start_idx, V_dt, V_num,
           E_mask, V_dispatch_mask, E_pt_dif, E_dt_dif, cou, A, W_node,
           W_edge, cou_table):
    del V_reach_mask
    B, N, _ = V.shape
    T = start_idx.shape[1]
    NN = N * N
    H = W_node.shape[1]
    f32 = jnp.float32

    sidx = start_idx.astype(jnp.int32)
    sd_f = E_sd.reshape(B, 1, NN)
    ed_f = E_ed.reshape(B, 1, NN)
    pt_f = E_pt_dif.reshape(B, 1, NN)
    dt_f = E_dt_dif.reshape(B, 1, NN)
    m_f = E_mask.reshape(B, T, 1, NN)
    a_f = A.reshape(B, T, 1, NN)
    vpt_f = V_pt.reshape(B, 1, N)
    vdt_f = V_dt.reshape(B, 1, N)
    vft_f = V_ft.reshape(B, 1, N)
    vnum_f = V_num.reshape(B, T, 1, N)
    dm_f = V_dispatch_mask.reshape(B, T, 1, N)

    BB = _BB
    per_bf = pl.BlockSpec((BB, 1, NN), lambda b, s_ref: (b, 0, 0))
    per_btf = pl.BlockSpec((BB, T, 1, NN), lambda b, s_ref: (b, 0, 0, 0))
    per_bnat = pl.BlockSpec((BB, N, N), lambda b, s_ref: (b, 0, 0))
    per_bn = pl.BlockSpec((BB, 1, N), lambda b, s_ref: (b, 0, 0))
    per_btn = pl.BlockSpec((BB, T, 1, N), lambda b, s_ref: (b, 0, 0, 0))

    node_h, edge_r = pl.pallas_call(
        _main_body,
        grid_spec=pltpu.PrefetchScalarGridSpec(
            num_scalar_prefetch=1,
            grid=(B // BB,),
            in_specs=[
                per_bf, per_bf, per_bf, per_bf,
                per_btf, per_btf,
                per_bnat, per_bnat,
                pl.BlockSpec((BB, N, 3), lambda b, s_ref: (b, 0, 0)),
                per_bn, per_bn, per_bn,
                per_btn, per_btn,
                pl.BlockSpec((8, H), lambda b, s_ref: (0, 0)),
                pl.BlockSpec((5, H), lambda b, s_ref: (0, 0)),
            ],
            out_specs=[
                pl.BlockSpec((BB, T, N, H), lambda b, s_ref: (b, 0, 0, 0)),
                pl.BlockSpec(memory_space=pl.ANY),
            ],
            scratch_shapes=[
                pltpu.VMEM((BB * T, NN, H), f32),
                pltpu.SemaphoreType.DMA((BB * T,)),
            ],
        ),
        out_shape=[
            jax.ShapeDtypeStruct((B, T, N, H), f32),
            jax.ShapeDtypeStruct((B, T, NN, H), f32),
        ],
    )(sidx, sd_f, ed_f, pt_f, dt_f, m_f, a_f, E_sd, E_ed, V,
      vpt_f, vdt_f, vft_f, vnum_f, dm_f, W_node, W_edge)

    edge_h = edge_r.reshape(B, T, N, N, H)

    ids = cou[:, 0].astype(jnp.int32)
    E = _COU_EMBED
    cou3 = cou.reshape(B, 1, 4)
    embed_b = pl.pallas_call(
        _embed_body,
        grid_spec=pltpu.PrefetchScalarGridSpec(
            num_scalar_prefetch=1,
            grid=(B,),
            in_specs=[
                pl.BlockSpec(memory_space=pl.ANY),
                pl.BlockSpec((1, 1, 4), lambda b, ids_ref: (b, 0, 0)),
            ],
            out_specs=pl.BlockSpec((1, T, E + 3), lambda b, ids_ref: (b, 0, 0)),
            scratch_shapes=[
                pltpu.VMEM((E, 128), f32),
                pltpu.SemaphoreType.DMA,
            ],
        ),
        out_shape=jax.ShapeDtypeStruct((B, T, E + 3), f32),
    )(ids, cou_table.T, cou3)
    embed_cou = embed_b.reshape(B * T, E + 3)

    return (node_h, edge_h, embed_cou)


# speedup vs baseline: 1.2860x; 1.2860x over previous
import jax
import jax.numpy as jnp
from jax.experimental import pallas as pl
from jax.experimental.pallas import tpu as pltpu

_T = 13
_COU_EMBED = 32
_DN = (((0,), (0,)), ((), ()))


_BB = 2


def _main_body(sidx_ref,
               sd_ref, ed_ref, pt_ref, dt_ref, m_ref, a_ref,
               sdn_ref, edn_ref, vT_ref, vpt_ref, vdt_ref, vft_ref,
               vnum_ref, dm_ref, wn_ref, we_ref,
               node_out_ref, edge_out_ref):
    g = pl.program_id(0)
    n = node_out_ref.shape[2]
    f32 = jnp.float32

    we = we_ref[...]
    wn = wn_ref[...]
    iota_s = jax.lax.broadcasted_iota(jnp.int32, (n, 1), 0)
    iota_l = jax.lax.broadcasted_iota(jnp.int32, (1, n), 1)

    for bb in range(_BB):
        sd = sd_ref[bb]
        ed = ed_ref[bb]
        pt = pt_ref[bb]
        dt = dt_ref[bb]
        sdn = sdn_ref[bb]
        edn = edn_ref[bb]
        vT = jnp.transpose(vT_ref[bb])
        vpt = vpt_ref[bb]
        vdt = vdt_ref[bb]
        vft = vft_ref[bb]

        for t in range(_T):
            m = m_ref[bb, t]
            x = jnp.concatenate(
                [sd * m, ed * m, pt * m, dt * m, a_ref[bb, t]], axis=0)
            edge_out_ref[bb, t] = jax.lax.dot_general(
                x, we, _DN, preferred_element_type=f32)

            s = sidx_ref[g * _BB + bb, t]
            ohs = (iota_s == s).astype(f32)
            ohl = (iota_l == s).astype(f32)
            ed_row = jnp.sum(edn * ohs, axis=0, keepdims=True)
            sd_row = jnp.sum(sdn * ohs, axis=0, keepdims=True)
            t_c = jnp.sum(vft * ohl)
            dm = dm_ref[bb, t]
            xn = jnp.concatenate([
                vT,
                vpt - t_c,
                t_c - vdt,
                ed_row * dm,
                sd_row * dm,
                vnum_ref[bb, t],
            ], axis=0) * dm
            node_out_ref[bb, t] = jax.lax.dot_general(
                xn, wn, _DN, preferred_element_type=f32)


def _embed_body(ids_ref, table_hbm, cou_ref, out_ref, row_vmem, sem):
    b = pl.program_id(0)
    idx = ids_ref[b]
    base = pl.multiple_of((idx // 128) * 128, 128)
    cp = pltpu.make_async_copy(
        table_hbm.at[:, pl.ds(base, 128)], row_vmem, sem)
    cp.start()
    cp.wait()
    off = idx % 128
    ohl = (jax.lax.broadcasted_iota(jnp.int32, (1, 128), 1) == off
           ).astype(jnp.float32)
    row = jnp.sum(row_vmem[...] * ohl, axis=1, keepdims=True)
    t = out_ref.shape[1]
    out_ref[0, :, 0:_COU_EMBED] = jnp.broadcast_to(
        jnp.transpose(row), (t, _COU_EMBED))
    out_ref[0, :, _COU_EMBED:_COU_EMBED + 3] = jnp.broadcast_to(
        cou_ref[0][:, 1:4], (t, 3))


def kernel(E_ed, V, V_reach_mask, V_pt, E_sd, V_ft, start_idx, V_dt, V_num,
           E_mask, V_dispatch_mask, E_pt_dif, E_dt_dif, cou, A, W_node,
           W_edge, cou_table):
    del V_reach_mask
    B, N, _ = V.shape
    T = start_idx.shape[1]
    NN = N * N
    H = W_node.shape[1]
    f32 = jnp.float32

    sidx = start_idx.astype(jnp.int32)
    sd_f = E_sd.reshape(B, 1, NN)
    ed_f = E_ed.reshape(B, 1, NN)
    pt_f = E_pt_dif.reshape(B, 1, NN)
    dt_f = E_dt_dif.reshape(B, 1, NN)
    m_f = E_mask.reshape(B, T, 1, NN)
    a_f = A.reshape(B, T, 1, NN)
    vpt_f = V_pt.reshape(B, 1, N)
    vdt_f = V_dt.reshape(B, 1, N)
    vft_f = V_ft.reshape(B, 1, N)
    vnum_f = V_num.reshape(B, T, 1, N)
    dm_f = V_dispatch_mask.reshape(B, T, 1, N)

    BB = _BB
    per_bf = pl.BlockSpec((BB, 1, NN), lambda b, s_ref: (b, 0, 0))
    per_btf = pl.BlockSpec((BB, T, 1, NN), lambda b, s_ref: (b, 0, 0, 0))
    per_bnat = pl.BlockSpec((BB, N, N), lambda b, s_ref: (b, 0, 0))
    per_bn = pl.BlockSpec((BB, 1, N), lambda b, s_ref: (b, 0, 0))
    per_btn = pl.BlockSpec((BB, T, 1, N), lambda b, s_ref: (b, 0, 0, 0))

    node_h, edge_r = pl.pallas_call(
        _main_body,
        grid_spec=pltpu.PrefetchScalarGridSpec(
            num_scalar_prefetch=1,
            grid=(B // BB,),
            in_specs=[
                per_bf, per_bf, per_bf, per_bf,
                per_btf, per_btf,
                per_bnat, per_bnat,
                pl.BlockSpec((BB, N, 3), lambda b, s_ref: (b, 0, 0)),
                per_bn, per_bn, per_bn,
                per_btn, per_btn,
                pl.BlockSpec((8, H), lambda b, s_ref: (0, 0)),
                pl.BlockSpec((5, H), lambda b, s_ref: (0, 0)),
            ],
            out_specs=[
                pl.BlockSpec((BB, T, N, H), lambda b, s_ref: (b, 0, 0, 0)),
                pl.BlockSpec((BB, T, NN, H), lambda b, s_ref: (b, 0, 0, 0)),
            ],
        ),
        out_shape=[
            jax.ShapeDtypeStruct((B, T, N, H), f32),
            jax.ShapeDtypeStruct((B, T, NN, H), f32),
        ],
    )(sidx, sd_f, ed_f, pt_f, dt_f, m_f, a_f, E_sd, E_ed, V,
      vpt_f, vdt_f, vft_f, vnum_f, dm_f, W_node, W_edge)

    edge_h = edge_r.reshape(B, T, N, N, H)

    ids = cou[:, 0].astype(jnp.int32)
    E = _COU_EMBED
    cou3 = cou.reshape(B, 1, 4)
    embed_b = pl.pallas_call(
        _embed_body,
        grid_spec=pltpu.PrefetchScalarGridSpec(
            num_scalar_prefetch=1,
            grid=(B,),
            in_specs=[
                pl.BlockSpec(memory_space=pl.ANY),
                pl.BlockSpec((1, 1, 4), lambda b, ids_ref: (b, 0, 0)),
            ],
            out_specs=pl.BlockSpec((1, T, E + 3), lambda b, ids_ref: (b, 0, 0)),
            scratch_shapes=[
                pltpu.VMEM((E, 128), f32),
                pltpu.SemaphoreType.DMA,
            ],
        ),
        out_shape=jax.ShapeDtypeStruct((B, T, E + 3), f32),
    )(ids, cou_table.T, cou3)
    embed_cou = embed_b.reshape(B * T, E + 3)

    return (node_h, edge_h, embed_cou)


# speedup vs baseline: 1.4078x; 1.0947x over previous
import jax
import jax.numpy as jnp
from jax.experimental import pallas as pl
from jax.experimental.pallas import tpu as pltpu

_T = 13
_COU_EMBED = 32
_DN = (((0,), (0,)), ((), ()))
_BB = 2


def _main_body(sidx_ref, ids_ref,
               sd_ref, ed_ref, pt_ref, dt_ref, m_ref, a_ref,
               sdn_ref, edn_ref, vT_ref, vpt_ref, vdt_ref, vft_ref,
               vnum_ref, dm_ref, wn_ref, we_ref, table_hbm, cou_ref,
               node_out_ref, edge_out_ref, embed_out_ref,
               tscratch, tsem):
    g = pl.program_id(0)
    n = node_out_ref.shape[2]
    btot = ids_ref.shape[0]
    ecols = _COU_EMBED
    f32 = jnp.float32

    @pl.when(g == 0)
    def _start_embed_dmas():
        for b0 in range(btot):
            idx0 = ids_ref[b0]
            base0 = pl.multiple_of((idx0 // 128) * 128, 128)
            pltpu.make_async_copy(
                table_hbm.at[:, pl.ds(base0, 128)],
                tscratch.at[b0], tsem.at[b0]).start()

    we = we_ref[...]
    wn = wn_ref[...]
    iota_s = jax.lax.broadcasted_iota(jnp.int32, (n, 1), 0)
    iota_l = jax.lax.broadcasted_iota(jnp.int32, (1, n), 1)
    iota_128 = jax.lax.broadcasted_iota(jnp.int32, (1, 128), 1)

    for bb in range(_BB):
        sd = sd_ref[bb]
        ed = ed_ref[bb]
        pt = pt_ref[bb]
        dt = dt_ref[bb]
        sdn = sdn_ref[bb]
        edn = edn_ref[bb]
        vT = jnp.transpose(vT_ref[bb])
        vpt = vpt_ref[bb]
        vdt = vdt_ref[bb]
        vft = vft_ref[bb]

        for t in range(_T):
            m = m_ref[bb, t]
            x = jnp.concatenate(
                [sd * m, ed * m, pt * m, dt * m, a_ref[bb, t]], axis=0)
            edge_out_ref[bb, t] = jax.lax.dot_general(
                x, we, _DN, preferred_element_type=f32)

            s = sidx_ref[g * _BB + bb, t]
            ohs = (iota_s == s).astype(f32)
            ohl = (iota_l == s).astype(f32)
            ed_row = jnp.sum(edn * ohs, axis=0, keepdims=True)
            sd_row = jnp.sum(sdn * ohs, axis=0, keepdims=True)
            t_c = jnp.sum(vft * ohl)
            dm = dm_ref[bb, t]
            xn = jnp.concatenate([
                vT,
                vpt - t_c,
                t_c - vdt,
                ed_row * dm,
                sd_row * dm,
                vnum_ref[bb, t],
            ], axis=0) * dm
            node_out_ref[bb, t] = jax.lax.dot_general(
                xn, wn, _DN, preferred_element_type=f32)

        bdyn = g * _BB + bb
        idx = ids_ref[bdyn]
        base = pl.multiple_of((idx // 128) * 128, 128)
        pltpu.make_async_copy(
            table_hbm.at[:, pl.ds(base, 128)],
            tscratch.at[bdyn], tsem.at[bdyn]).wait()
        ohe = (iota_128 == idx % 128).astype(f32)
        row = jnp.sum(tscratch[bdyn] * ohe, axis=1, keepdims=True)
        row35 = jnp.concatenate(
            [jnp.transpose(row), cou_ref[bdyn][:, 1:4]], axis=1)
        embed_out_ref[bdyn] = jnp.broadcast_to(row35, (_T, ecols + 3))


def kernel(E_ed, V, V_reach_mask, V_pt, E_sd, V_ft, start_idx, V_dt, V_num,
           E_mask, V_dispatch_mask, E_pt_dif, E_dt_dif, cou, A, W_node,
           W_edge, cou_table):
    del V_reach_mask
    B, N, _ = V.shape
    T = start_idx.shape[1]
    NN = N * N
    H = W_node.shape[1]
    E = _COU_EMBED
    f32 = jnp.float32

    sidx = start_idx.astype(jnp.int32)
    ids = cou[:, 0].astype(jnp.int32)
    sd_f = E_sd.reshape(B, 1, NN)
    ed_f = E_ed.reshape(B, 1, NN)
    pt_f = E_pt_dif.reshape(B, 1, NN)
    dt_f = E_dt_dif.reshape(B, 1, NN)
    m_f = E_mask.reshape(B, T, 1, NN)
    a_f = A.reshape(B, T, 1, NN)
    vpt_f = V_pt.reshape(B, 1, N)
    vdt_f = V_dt.reshape(B, 1, N)
    vft_f = V_ft.reshape(B, 1, N)
    vnum_f = V_num.reshape(B, T, 1, N)
    dm_f = V_dispatch_mask.reshape(B, T, 1, N)
    cou3 = cou.reshape(B, 1, 4)

    BB = _BB
    per_bf = pl.BlockSpec((BB, 1, NN), lambda b, s_ref, i_ref: (b, 0, 0))
    per_btf = pl.BlockSpec((BB, T, 1, NN),
                           lambda b, s_ref, i_ref: (b, 0, 0, 0))
    per_bnat = pl.BlockSpec((BB, N, N), lambda b, s_ref, i_ref: (b, 0, 0))
    per_bn = pl.BlockSpec((BB, 1, N), lambda b, s_ref, i_ref: (b, 0, 0))
    per_btn = pl.BlockSpec((BB, T, 1, N),
                           lambda b, s_ref, i_ref: (b, 0, 0, 0))

    node_h, edge_r, embed_b = pl.pallas_call(
        _main_body,
        grid_spec=pltpu.PrefetchScalarGridSpec(
            num_scalar_prefetch=2,
            grid=(B // BB,),
            in_specs=[
                per_bf, per_bf, per_bf, per_bf,
                per_btf, per_btf,
                per_bnat, per_bnat,
                pl.BlockSpec((BB, N, 3),
                             lambda b, s_ref, i_ref: (b, 0, 0)),
                per_bn, per_bn, per_bn,
                per_btn, per_btn,
                pl.BlockSpec((8, H), lambda b, s_ref, i_ref: (0, 0)),
                pl.BlockSpec((5, H), lambda b, s_ref, i_ref: (0, 0)),
                pl.BlockSpec(memory_space=pl.ANY),
                pl.BlockSpec((B, 1, 4),
                             lambda b, s_ref, i_ref: (0, 0, 0)),
            ],
            out_specs=[
                pl.BlockSpec((BB, T, N, H),
                             lambda b, s_ref, i_ref: (b, 0, 0, 0)),
                pl.BlockSpec((BB, T, NN, H),
                             lambda b, s_ref, i_ref: (b, 0, 0, 0)),
                pl.BlockSpec((B, T, E + 3),
                             lambda b, s_ref, i_ref: (0, 0, 0)),
            ],
            scratch_shapes=[
                pltpu.VMEM((B, E, 128), f32),
                pltpu.SemaphoreType.DMA((B,)),
            ],
        ),
        out_shape=[
            jax.ShapeDtypeStruct((B, T, N, H), f32),
            jax.ShapeDtypeStruct((B, T, NN, H), f32),
            jax.ShapeDtypeStruct((B, T, E + 3), f32),
        ],
    )(sidx, ids, sd_f, ed_f, pt_f, dt_f, m_f, a_f, E_sd, E_ed, V,
      vpt_f, vdt_f, vft_f, vnum_f, dm_f, W_node, W_edge, cou_table.T, cou3)

    edge_h = edge_r.reshape(B, T, N, N, H)
    embed_cou = embed_b.reshape(B * T, E + 3)

    return (node_h, edge_h, embed_cou)
